# baseline (speedup 1.0000x reference)
import functools

import jax
import jax.numpy as jnp
from jax import lax
from jax.experimental import pallas as pl
from jax.experimental.pallas import tpu as pltpu
from jax.experimental.pallas import tpu_sc as plsc

VOCAB = 1000000
CONTENT_DIM = 64
TIME_DIM = 16
OUT_DIM = CONTENT_DIM + TIME_DIM
BATCH = 4096
SEQ = 200

_INFO = plsc.get_sparse_core_info()
NC, NS = _INFO.num_cores, _INFO.num_subcores
NW = NC * NS

CHUNK = 256
BT = BATCH // CHUNK
DT = OUT_DIM // 8
N_CHUNKS = SEQ * BT
CH_PER_W = N_CHUNKS // NW


QPAD = 129

TCW = 1024


def _tc_relayout_body(ct_ref, tt_ref, crm_ref, trm_ref):
    crm_ref[...] = ct_ref[...].T
    trm_ref[...] = tt_ref[...].T


def _gather_body(ids_hbm, content_hbm, time_hbm, out_hbm,
                 idx0, idx1, rc0, rc1, rt0, rt1, qb0, qb1,
                 sem_i0, sem_i1, sem_c0, sem_c1, sem_t0, sem_t1,
                 sem_o0, sem_o1):
    wid = lax.axis_index("s") * NC + lax.axis_index("c")
    base = wid * CH_PER_W
    end = base + CH_PER_W
    lane = lax.iota(jnp.int32, 16)
    didx_c = [k * 16 + lane for k in range(CONTENT_DIM // 16)]
    didx_t = CONTENT_DIM + lane

    idx_v = [idx0, idx1]
    rows_c = [rc0, rc1]
    rows_t = [rt0, rt1]
    qbuf = [qb0, qb1]
    sem_i = [sem_i0, sem_i1]
    sem_c = [sem_c0, sem_c1]
    sem_t = [sem_t0, sem_t1]
    sem_o = [sem_o0, sem_o1]

    def out_copies(c, slot):
        s = c // BT
        b2 = c % BT
        return [
            pltpu.make_async_copy(
                qbuf[slot].at[:, pl.ds(dt * 8, 8), pl.ds(0, 128)],
                out_hbm.at[s, dt, pl.ds(2 * b2, 2)], sem_o[slot])
            for dt in range(DT)
        ]

    def idx_copy(c, slot):
        return pltpu.make_async_copy(
            ids_hbm.at[pl.ds(2 * c, 2), :], idx_v[slot], sem_i[slot])

    def gathers(slot):
        cps = []
        for h in range(2):
            cps.append(pltpu.make_async_copy(
                content_hbm.at[idx_v[slot].at[h]],
                rows_c[slot].at[pl.ds(h * 128, 128), :], sem_c[slot]))
            cps.append(pltpu.make_async_copy(
                time_hbm.at[idx_v[slot].at[h]],
                rows_t[slot].at[pl.ds(h * 128, 128), :], sem_t[slot]))
        return cps

    idx_copy(base, 0).start()
    idx_copy(base + 1, 1).start()
    idx_copy(base, 0).wait()
    for g in gathers(0):
        g.start()

    def pair_body(i, carry):
        c0 = base + 2 * i
        for b in range(2):
            c = c0 + b
            slot, other = b, 1 - b

            for g in gathers(slot):
                g.wait()

            @pl.when(c + 2 < end)
            def _idx_pf():
                idx_copy(c + 2, slot).start()

            @pl.when(c + 1 < end)
            def _gather_pf():
                idx_copy(c + 1, other).wait()
                for g in gathers(other):
                    g.start()

            @pl.when(c >= base + 2)
            def _out_drain():
                for o in out_copies(c - 2, slot):
                    o.wait()

            def tr_body(j, carry2):
                jh = jnp.full((16,), j // 128, jnp.int32)
                jv = jnp.full((16,), j % 128, jnp.int32)
                for k in range(CONTENT_DIM // 16):
                    x = rows_c[slot][j, pl.ds(k * 16, 16)]
                    plsc.store_scatter(qbuf[slot], [jh, didx_c[k], jv], x)
                x = rows_t[slot][j, pl.ds(0, 16)]
                plsc.store_scatter(qbuf[slot], [jh, didx_t, jv], x)
                return carry2

            lax.fori_loop(0, CHUNK, tr_body, 0, unroll=8)

            for o in out_copies(c, slot):
                o.start()
        return carry

    lax.fori_loop(0, CH_PER_W // 2, pair_body, 0)
    for o in out_copies(end - 2, 0):
        o.wait()
    for o in out_copies(end - 1, 1):
        o.wait()


@jax.jit
def kernel(token_ids, content_table, time_table):
    ids = token_ids.T.reshape(BATCH * SEQ // 128, 128)
    mesh = plsc.VectorSubcoreMesh(core_axis_name="c", subcore_axis_name="s")

    content_rm, time_rm = pl.pallas_call(
        _tc_relayout_body,
        out_shape=(
            jax.ShapeDtypeStruct((VOCAB, CONTENT_DIM), jnp.float32),
            jax.ShapeDtypeStruct((VOCAB, TIME_DIM), jnp.float32),
        ),
        grid=(pl.cdiv(VOCAB, TCW),),
        in_specs=[
            pl.BlockSpec((CONTENT_DIM, TCW), lambda i: (0, i)),
            pl.BlockSpec((TIME_DIM, TCW), lambda i: (0, i)),
        ],
        out_specs=(
            pl.BlockSpec((TCW, CONTENT_DIM), lambda i: (i, 0)),
            pl.BlockSpec((TCW, TIME_DIM), lambda i: (i, 0)),
        ),
    )(content_table.T, time_table.T)

    q = pl.kernel(
        _gather_body,
        out_type=jax.ShapeDtypeStruct((SEQ, DT, 2 * BT, 8, 128), jnp.float32),
        mesh=mesh,
        scratch_types=[
            pltpu.VMEM((2, 128), jnp.int32),
            pltpu.VMEM((2, 128), jnp.int32),
            pltpu.VMEM((CHUNK, CONTENT_DIM), jnp.float32),
            pltpu.VMEM((CHUNK, CONTENT_DIM), jnp.float32),
            pltpu.VMEM((CHUNK, TIME_DIM), jnp.float32),
            pltpu.VMEM((CHUNK, TIME_DIM), jnp.float32),
            pltpu.VMEM((2, OUT_DIM, QPAD), jnp.float32),
            pltpu.VMEM((2, OUT_DIM, QPAD), jnp.float32),
        ] + [pltpu.SemaphoreType.DMA] * 8,
        compiler_params=pltpu.CompilerParams(
            use_tc_tiling_on_sc=False, needs_layout_passes=False),
    )(ids, content_rm, time_rm)
    return q.transpose(2, 4, 0, 1, 3).reshape(BATCH, SEQ, OUT_DIM)


# speedup vs baseline: 1.8034x; 1.8034x over previous
import functools

import jax
import jax.numpy as jnp
from jax import lax
from jax.experimental import pallas as pl
from jax.experimental.pallas import tpu as pltpu
from jax.experimental.pallas import tpu_sc as plsc

VOCAB = 1000000
CONTENT_DIM = 64
TIME_DIM = 16
OUT_DIM = CONTENT_DIM + TIME_DIM
BATCH = 4096
SEQ = 200

_INFO = plsc.get_sparse_core_info()
NC, NS = _INFO.num_cores, _INFO.num_subcores
NW = NC * NS

CHUNK = 256
BT = BATCH // CHUNK
DT = OUT_DIM // 8
N_CHUNKS = SEQ * BT
CH_PER_W = N_CHUNKS // NW

QPAD = 129


NT_FULL = VOCAB // 128
NT_W = (NT_FULL + NW - 1) // NW
TAIL0 = NT_FULL * 128


def _fuse_body(ct_hbm, tt_hbm, tailc_hbm, tailt_hbm, frm_hbm,
               cin0, cin1, tin0, tin1, fout0, fout1,
               sem_i0, sem_i1, sem_o0, sem_o1):
    wid = lax.axis_index("s") * NC + lax.axis_index("c")
    lane = lax.iota(jnp.int32, 16)

    cin = [cin0, cin1]
    tin = [tin0, tin1]
    fout = [fout0, fout1]
    sem_i = [sem_i0, sem_i1]
    sem_o = [sem_o0, sem_o1]

    def tile_of(u):
        return wid + NW * u

    def in_copies(tt, slot):
        cps = [
            pltpu.make_async_copy(
                ct_hbm.at[pl.ds(dt * 8, 8), pl.ds(tt * 128, 128)],
                cin[slot].at[pl.ds(dt * 8, 8), :], sem_i[slot])
            for dt in range(CONTENT_DIM // 8)
        ]
        cps += [
            pltpu.make_async_copy(
                tt_hbm.at[pl.ds(dt * 8, 8), pl.ds(tt * 128, 128)],
                tin[slot].at[pl.ds(dt * 8, 8), :], sem_i[slot])
            for dt in range(TIME_DIM // 8)
        ]
        return cps

    def out_copy(tt, slot):
        return pltpu.make_async_copy(
            fout[slot], frm_hbm.at[pl.ds(tt * 80, 80), :], sem_o[slot])

    for cp in in_copies(tile_of(0), 0):
        cp.start()

    jvec = [16 * jb + lane for jb in range(8)]
    lane80 = lane * 80

    def transpose(slot):
        def r_body(r, carry):
            dv0 = (lane + r) & 15
            w0 = lane80 + dv0
            for kb in range(CONTENT_DIM // 16):
                wk = w0 + 16 * kb
                rowk = lax.shift_right_logical(wk, 7)
                colk = lax.bitwise_and(wk, 127)
                dv = dv0 + 16 * kb
                for jb in range(8):
                    x = plsc.load_gather(cin[slot], [dv, jvec[jb]])
                    plsc.store_scatter(
                        fout[slot].at[pl.ds(10 * jb, 10), :], [rowk, colk], x)
            wt = w0 + CONTENT_DIM
            rowt = lax.shift_right_logical(wt, 7)
            colt = lax.bitwise_and(wt, 127)
            for jb in range(8):
                x = plsc.load_gather(tin[slot], [dv0, jvec[jb]])
                plsc.store_scatter(
                    fout[slot].at[pl.ds(10 * jb, 10), :], [rowt, colt], x)
            return carry

        lax.fori_loop(0, 16, r_body, 0)

    def pair(i, carry):
        for b in range(2):
            u = 2 * i + b
            slot, other = b, 1 - b
            tt = tile_of(u)

            @pl.when(tt < NT_FULL)
            def _step():
                for cp in in_copies(tt, slot):
                    cp.wait()

                @pl.when(tile_of(u + 1) < NT_FULL)
                def _pf():
                    for cp in in_copies(tile_of(u + 1), other):
                        cp.start()

                @pl.when(u >= 2)
                def _drain():
                    out_copy(tile_of(u - 2), slot).wait()

                transpose(slot)
                out_copy(tt, slot).start()
        return carry

    lax.fori_loop(0, (NT_W + 2) // 2, pair, 0)
    out_copy(tile_of(0), 0).wait()
    out_copy(tile_of(0), 1).wait()

    @pl.when(wid == NW - 1)
    def _tail():
        pltpu.sync_copy(tailc_hbm, cin[0].at[pl.ds(0, 32), :])
        pltpu.sync_copy(tailt_hbm, tin[0].at[pl.ds(0, 8), :])

        def t_body(t, carry):
            for k in range(CONTENT_DIM // 16):
                fc = t * 64 + k * 16 + lane
                x = plsc.load_gather(
                    cin[0], [lax.shift_right_logical(fc, 7),
                             lax.bitwise_and(fc, 127)])
                w = t * 80 + k * 16 + lane
                plsc.store_scatter(
                    fout[0], [lax.shift_right_logical(w, 7),
                              lax.bitwise_and(w, 127)], x)
            ft = t * 16 + lane
            x = plsc.load_gather(
                tin[0], [lax.shift_right_logical(ft, 7),
                         lax.bitwise_and(ft, 127)])
            w = t * 80 + CONTENT_DIM + lane
            plsc.store_scatter(
                fout[0], [lax.shift_right_logical(w, 7),
                          lax.bitwise_and(w, 127)], x)
            return carry

        lax.fori_loop(0, 64, t_body, 0)
        pltpu.sync_copy(fout[0].at[pl.ds(0, 40), :],
                        frm_hbm.at[pl.ds(TAIL0 * 80 // 128, 40), :])


def _gather_body(ids_hbm, fused_hbm, out_hbm,
                 idx0, idx1, rf0, rf1, qb0, qb1,
                 sem_i0, sem_i1, sem_g0, sem_g1, sem_o0, sem_o1):
    wid = lax.axis_index("s") * NC + lax.axis_index("c")
    base = wid * CH_PER_W
    end = base + CH_PER_W
    lane = lax.iota(jnp.int32, 16)
    didx = [k * 16 + lane for k in range(OUT_DIM // 16)]

    idx_v = [idx0, idx1]
    rows_f = [rf0, rf1]
    qbuf = [qb0, qb1]
    sem_i = [sem_i0, sem_i1]
    sem_g = [sem_g0, sem_g1]
    sem_o = [sem_o0, sem_o1]

    def out_copies(c, slot):
        s = c // BT
        b2 = c % BT
        return [
            pltpu.make_async_copy(
                qbuf[slot].at[:, pl.ds(dt * 8, 8), pl.ds(0, 128)],
                out_hbm.at[s, dt, pl.ds(2 * b2, 2)], sem_o[slot])
            for dt in range(DT)
        ]

    def idx_copy(c, slot):
        return pltpu.make_async_copy(
            ids_hbm.at[pl.ds(2 * c, 2), :], idx_v[slot], sem_i[slot])

    def gathers(slot):
        return [
            pltpu.make_async_copy(
                fused_hbm.at[idx_v[slot].at[h]],
                rows_f[slot].at[pl.ds(h * 128, 128), :], sem_g[slot])
            for h in range(2)
        ]

    idx_copy(base, 0).start()
    idx_copy(base + 1, 1).start()
    idx_copy(base, 0).wait()
    for g in gathers(0):
        g.start()

    def pair_body(i, carry):
        c0 = base + 2 * i
        for b in range(2):
            c = c0 + b
            slot, other = b, 1 - b

            for g in gathers(slot):
                g.wait()

            @pl.when(c + 2 < end)
            def _idx_pf():
                idx_copy(c + 2, slot).start()

            @pl.when(c + 1 < end)
            def _gather_pf():
                idx_copy(c + 1, other).wait()
                for g in gathers(other):
                    g.start()

            @pl.when(c >= base + 2)
            def _out_drain():
                for o in out_copies(c - 2, slot):
                    o.wait()

            def tr_body(j, carry2):
                jh = jnp.full((16,), j // 128, jnp.int32)
                jv = jnp.full((16,), j % 128, jnp.int32)
                for k in range(OUT_DIM // 16):
                    x = rows_f[slot][j, pl.ds(k * 16, 16)]
                    plsc.store_scatter(qbuf[slot], [jh, didx[k], jv], x)
                return carry2

            lax.fori_loop(0, CHUNK, tr_body, 0, unroll=8)

            for o in out_copies(c, slot):
                o.start()
        return carry

    lax.fori_loop(0, CH_PER_W // 2, pair_body, 0)
    for o in out_copies(end - 2, 0):
        o.wait()
    for o in out_copies(end - 1, 1):
        o.wait()


@jax.jit
def kernel(token_ids, content_table, time_table):
    ids = token_ids.T.reshape(BATCH * SEQ // 128, 128)
    mesh = plsc.VectorSubcoreMesh(core_axis_name="c", subcore_axis_name="s")

    frm = pl.kernel(
        _fuse_body,
        out_type=jax.ShapeDtypeStruct((VOCAB * OUT_DIM // 128, 128),
                                      jnp.float32),
        mesh=mesh,
        scratch_types=[
            pltpu.VMEM((CONTENT_DIM, 128), jnp.float32),
            pltpu.VMEM((CONTENT_DIM, 128), jnp.float32),
            pltpu.VMEM((TIME_DIM, 128), jnp.float32),
            pltpu.VMEM((TIME_DIM, 128), jnp.float32),
            pltpu.VMEM((OUT_DIM, 128), jnp.float32),
            pltpu.VMEM((OUT_DIM, 128), jnp.float32),
        ] + [pltpu.SemaphoreType.DMA] * 4,
        compiler_params=pltpu.CompilerParams(
            use_tc_tiling_on_sc=True, needs_layout_passes=False),
    )(
        content_table.T,
        time_table.T,
        content_table[TAIL0:].reshape(32, 128),
        time_table[TAIL0:].reshape(8, 128),
    )
    fused = frm.reshape(VOCAB, OUT_DIM)

    q = pl.kernel(
        _gather_body,
        out_type=jax.ShapeDtypeStruct((SEQ, DT, 2 * BT, 8, 128), jnp.float32),
        mesh=mesh,
        scratch_types=[
            pltpu.VMEM((2, 128), jnp.int32),
            pltpu.VMEM((2, 128), jnp.int32),
            pltpu.VMEM((CHUNK, OUT_DIM), jnp.float32),
            pltpu.VMEM((CHUNK, OUT_DIM), jnp.float32),
            pltpu.VMEM((2, OUT_DIM, QPAD), jnp.float32),
            pltpu.VMEM((2, OUT_DIM, QPAD), jnp.float32),
        ] + [pltpu.SemaphoreType.DMA] * 6,
        compiler_params=pltpu.CompilerParams(
            use_tc_tiling_on_sc=False, needs_layout_passes=False),
    )(ids, fused)
    return q.transpose(2, 4, 0, 1, 3).reshape(BATCH, SEQ, OUT_DIM)


# speedup vs baseline: 1.8186x; 1.0084x over previous
import functools

import jax
import jax.numpy as jnp
from jax import lax
from jax.experimental import pallas as pl
from jax.experimental.pallas import tpu as pltpu
from jax.experimental.pallas import tpu_sc as plsc

VOCAB = 1000000
CONTENT_DIM = 64
TIME_DIM = 16
OUT_DIM = CONTENT_DIM + TIME_DIM
BATCH = 4096
SEQ = 200

_INFO = plsc.get_sparse_core_info()
NC, NS = _INFO.num_cores, _INFO.num_subcores
NW = NC * NS

CHUNK = 256
BT = BATCH // CHUNK
DT = OUT_DIM // 8
N_CHUNKS = SEQ * BT
CH_PER_W = N_CHUNKS // NW

QPAD = 129


NT_FULL = VOCAB // 128
NT_W = (NT_FULL + NW - 1) // NW
TAIL0 = NT_FULL * 128


def _fuse_body(ct_hbm, tt_hbm, tailc_hbm, tailt_hbm, frm_hbm,
               cin0, cin1, tin0, tin1, fout0, fout1,
               sem_i0, sem_i1, sem_o0, sem_o1):
    wid = lax.axis_index("s") * NC + lax.axis_index("c")
    lane = lax.iota(jnp.int32, 16)

    cin = [cin0, cin1]
    tin = [tin0, tin1]
    fout = [fout0, fout1]
    sem_i = [sem_i0, sem_i1]
    sem_o = [sem_o0, sem_o1]

    def tile_of(u):
        return wid + NW * u

    def in_copies(tt, slot):
        return [
            pltpu.make_async_copy(
                ct_hbm.at[:, pl.ds(tt * 128, 128)], cin[slot], sem_i[slot]),
            pltpu.make_async_copy(
                tt_hbm.at[:, pl.ds(tt * 128, 128)], tin[slot], sem_i[slot]),
        ]

    def out_copy(tt, slot):
        return pltpu.make_async_copy(
            fout[slot], frm_hbm.at[pl.ds(tt * 80, 80), :], sem_o[slot])

    for cp in in_copies(tile_of(0), 0):
        cp.start()

    jvec = [16 * jb + lane for jb in range(8)]
    lane80 = lane * 80

    def transpose(slot):
        def r_body(r, carry):
            dv0 = (lane + r) & 15
            w0 = lane80 + dv0
            for kb in range(CONTENT_DIM // 16):
                wk = w0 + 16 * kb
                rowk = lax.shift_right_logical(wk, 7)
                colk = lax.bitwise_and(wk, 127)
                dv = dv0 + 16 * kb
                for jb in range(8):
                    x = plsc.load_gather(cin[slot], [dv, jvec[jb]])
                    plsc.store_scatter(
                        fout[slot].at[pl.ds(10 * jb, 10), :], [rowk, colk], x)
            wt = w0 + CONTENT_DIM
            rowt = lax.shift_right_logical(wt, 7)
            colt = lax.bitwise_and(wt, 127)
            for jb in range(8):
                x = plsc.load_gather(tin[slot], [dv0, jvec[jb]])
                plsc.store_scatter(
                    fout[slot].at[pl.ds(10 * jb, 10), :], [rowt, colt], x)
            return carry

        lax.fori_loop(0, 16, r_body, 0)

    def pair(i, carry):
        for b in range(2):
            u = 2 * i + b
            slot, other = b, 1 - b
            tt = tile_of(u)

            @pl.when(tt < NT_FULL)
            def _step():
                for cp in in_copies(tt, slot):
                    cp.wait()

                @pl.when(tile_of(u + 1) < NT_FULL)
                def _pf():
                    for cp in in_copies(tile_of(u + 1), other):
                        cp.start()

                @pl.when(u >= 2)
                def _drain():
                    out_copy(tile_of(u - 2), slot).wait()

                transpose(slot)
                out_copy(tt, slot).start()
        return carry

    lax.fori_loop(0, (NT_W + 2) // 2, pair, 0)
    out_copy(tile_of(0), 0).wait()
    out_copy(tile_of(0), 1).wait()

    @pl.when(wid == NW - 1)
    def _tail():
        pltpu.sync_copy(tailc_hbm, cin[0].at[pl.ds(0, 32), :])
        pltpu.sync_copy(tailt_hbm, tin[0].at[pl.ds(0, 8), :])

        def t_body(t, carry):
            for k in range(CONTENT_DIM // 16):
                fc = t * 64 + k * 16 + lane
                x = plsc.load_gather(
                    cin[0], [lax.shift_right_logical(fc, 7),
                             lax.bitwise_and(fc, 127)])
                w = t * 80 + k * 16 + lane
                plsc.store_scatter(
                    fout[0], [lax.shift_right_logical(w, 7),
                              lax.bitwise_and(w, 127)], x)
            ft = t * 16 + lane
            x = plsc.load_gather(
                tin[0], [lax.shift_right_logical(ft, 7),
                         lax.bitwise_and(ft, 127)])
            w = t * 80 + CONTENT_DIM + lane
            plsc.store_scatter(
                fout[0], [lax.shift_right_logical(w, 7),
                          lax.bitwise_and(w, 127)], x)
            return carry

        lax.fori_loop(0, 64, t_body, 0)
        pltpu.sync_copy(fout[0].at[pl.ds(0, 40), :],
                        frm_hbm.at[pl.ds(TAIL0 * 80 // 128, 40), :])


def _gather_body(ids_hbm, fused_hbm, out_hbm,
                 idx0, idx1, rf0, rf1, qb0, qb1,
                 sem_i0, sem_i1, sem_g0, sem_g1, sem_o0, sem_o1):
    wid = lax.axis_index("s") * NC + lax.axis_index("c")
    base = wid * CH_PER_W
    end = base + CH_PER_W
    lane = lax.iota(jnp.int32, 16)
    didx = [k * 16 + lane for k in range(OUT_DIM // 16)]

    idx_v = [idx0, idx1]
    rows_f = [rf0, rf1]
    qbuf = [qb0, qb1]
    sem_i = [sem_i0, sem_i1]
    sem_g = [sem_g0, sem_g1]
    sem_o = [sem_o0, sem_o1]

    def out_copies(c, slot):
        s = c // BT
        b2 = c % BT
        return [
            pltpu.make_async_copy(
                qbuf[slot].at[:, pl.ds(dt * 8, 8), pl.ds(0, 128)],
                out_hbm.at[s, dt, pl.ds(2 * b2, 2)], sem_o[slot])
            for dt in range(DT)
        ]

    def idx_copy(c, slot):
        return pltpu.make_async_copy(
            ids_hbm.at[pl.ds(2 * c, 2), :], idx_v[slot], sem_i[slot])

    def gathers(slot):
        return [
            pltpu.make_async_copy(
                fused_hbm.at[idx_v[slot].at[h]],
                rows_f[slot].at[pl.ds(h * 128, 128), :], sem_g[slot])
            for h in range(2)
        ]

    idx_copy(base, 0).start()
    idx_copy(base + 1, 1).start()
    idx_copy(base, 0).wait()
    for g in gathers(0):
        g.start()

    def pair_body(i, carry):
        c0 = base + 2 * i
        for b in range(2):
            c = c0 + b
            slot, other = b, 1 - b

            for g in gathers(slot):
                g.wait()

            @pl.when(c + 2 < end)
            def _idx_pf():
                idx_copy(c + 2, slot).start()

            @pl.when(c + 1 < end)
            def _gather_pf():
                idx_copy(c + 1, other).wait()
                for g in gathers(other):
                    g.start()

            @pl.when(c >= base + 2)
            def _out_drain():
                for o in out_copies(c - 2, slot):
                    o.wait()

            def tr_body(j, carry2):
                jh = jnp.full((16,), j // 128, jnp.int32)
                jv = jnp.full((16,), j % 128, jnp.int32)
                for k in range(OUT_DIM // 16):
                    x = rows_f[slot][j, pl.ds(k * 16, 16)]
                    plsc.store_scatter(qbuf[slot], [jh, didx[k], jv], x)
                return carry2

            lax.fori_loop(0, CHUNK, tr_body, 0, unroll=8)

            for o in out_copies(c, slot):
                o.start()
        return carry

    lax.fori_loop(0, CH_PER_W // 2, pair_body, 0)
    for o in out_copies(end - 2, 0):
        o.wait()
    for o in out_copies(end - 1, 1):
        o.wait()


@jax.jit
def kernel(token_ids, content_table, time_table):
    ids = token_ids.T.reshape(BATCH * SEQ // 128, 128)
    mesh = plsc.VectorSubcoreMesh(core_axis_name="c", subcore_axis_name="s")

    frm = pl.kernel(
        _fuse_body,
        out_type=jax.ShapeDtypeStruct((VOCAB * OUT_DIM // 128, 128),
                                      jnp.float32),
        mesh=mesh,
        scratch_types=[
            pltpu.VMEM((CONTENT_DIM, 128), jnp.float32),
            pltpu.VMEM((CONTENT_DIM, 128), jnp.float32),
            pltpu.VMEM((TIME_DIM, 128), jnp.float32),
            pltpu.VMEM((TIME_DIM, 128), jnp.float32),
            pltpu.VMEM((OUT_DIM, 128), jnp.float32),
            pltpu.VMEM((OUT_DIM, 128), jnp.float32),
        ] + [pltpu.SemaphoreType.DMA] * 4,
        compiler_params=pltpu.CompilerParams(
            use_tc_tiling_on_sc=True, needs_layout_passes=False),
    )(
        content_table.T,
        time_table.T,
        content_table[TAIL0:].reshape(32, 128),
        time_table[TAIL0:].reshape(8, 128),
    )
    fused = frm.reshape(VOCAB, OUT_DIM)

    q = pl.kernel(
        _gather_body,
        out_type=jax.ShapeDtypeStruct((SEQ, DT, 2 * BT, 8, 128), jnp.float32),
        mesh=mesh,
        scratch_types=[
            pltpu.VMEM((2, 128), jnp.int32),
            pltpu.VMEM((2, 128), jnp.int32),
            pltpu.VMEM((CHUNK, OUT_DIM), jnp.float32),
            pltpu.VMEM((CHUNK, OUT_DIM), jnp.float32),
            pltpu.VMEM((2, OUT_DIM, QPAD), jnp.float32),
            pltpu.VMEM((2, OUT_DIM, QPAD), jnp.float32),
        ] + [pltpu.SemaphoreType.DMA] * 6,
        compiler_params=pltpu.CompilerParams(
            use_tc_tiling_on_sc=False, needs_layout_passes=False),
    )(ids, fused)
    return q.transpose(2, 4, 0, 1, 3).reshape(BATCH, SEQ, OUT_DIM)
